# Initial kernel scaffold; baseline (speedup 1.0000x reference)
import jax
import jax.numpy as jnp
from jax.experimental import pallas as pl


def kernel(input_ids, alpha):
    raise NotImplementedError("write your pallas kernel here")


# speedup vs baseline: 5.4261x; 5.4261x over previous
import functools

import jax
import jax.numpy as jnp
from jax import lax
from jax.experimental import pallas as pl
from jax.experimental.pallas import tpu as pltpu
from jax.experimental.pallas import tpu_sc as plsc

_N = 3
_V = 100000
_B = 32
_L = 4096
_W = _L - _N + 1
_LANES = 16
_NC, _NS = 2, 16
_CHUNKS = (_W + _LANES - 1) // _LANES
_ROWBUF = _CHUNKS * _LANES + _LANES

_mesh = plsc.VectorSubcoreMesh(
    core_axis_name="c", subcore_axis_name="s", num_cores=_NC, num_subcores=_NS
)


@functools.partial(
    pl.kernel,
    out_type=[
        jax.ShapeDtypeStruct((_B, _V), jnp.float32),
        jax.ShapeDtypeStruct((_B, _LANES), jnp.float32),
    ],
    mesh=_mesh,
    scratch_types=[
        pltpu.VMEM((_ROWBUF,), jnp.int32),
        pltpu.VMEM((_V,), jnp.float32),
        pltpu.VMEM((_LANES,), jnp.int32),
        pltpu.VMEM((_LANES,), jnp.int32),
        pltpu.VMEM((_LANES,), jnp.float32),
        pltpu.VMEM((_LANES,), jnp.int32),
    ],
    compiler_params=pltpu.CompilerParams(needs_layout_passes=False),
)
def _sc_hist(ids_hbm, t1_hbm, t2_hbm, cnt_hbm, m_hbm, row_v, cnt_v, t1_v, t2_v, m_v, macc_v):
    b = lax.axis_index("s") * _NC + lax.axis_index("c")

    pltpu.sync_copy(ids_hbm.at[b], row_v.at[pl.ds(0, _L)])
    pltpu.sync_copy(t1_hbm.at[b], t1_v)
    pltpu.sync_copy(t2_hbm.at[b], t2_v)

    t1 = t1_v[...]
    t2 = t2_v[...]
    iota = lax.broadcasted_iota(jnp.int32, (_LANES,), 0)

    def _match(base):
        a = row_v[pl.ds(base, _LANES)]
        bb = row_v[pl.ds(base + 1, _LANES)]
        return (a == t1) & (bb == t2) & ((base + iota) < _W)

    def count_body(i, acc):
        return acc + plsc.all_reduce_population_count(_match(i * _LANES))

    acc = lax.fori_loop(0, _CHUNKS, count_body, jnp.zeros((_LANES,), jnp.int32))

    macc_v[...] = acc
    m = acc[0]
    m_v[...] = acc.astype(jnp.float32)
    pltpu.sync_copy(m_v, m_hbm.at[b])

    zeros16 = jnp.zeros((_LANES,), jnp.float32)
    ones16 = jnp.ones((_LANES,), jnp.float32)

    def zero_body(j, carry):
        cnt_v[pl.ds(j * _LANES, _LANES)] = zeros16
        return carry

    def scat_body(i, carry):
        base = i * _LANES
        match = _match(base)
        nx = jnp.where(match, row_v[pl.ds(base + 2, _LANES)], 0)
        for k in range(_LANES):
            plsc.addupdate_scatter(cnt_v, [nx], ones16, mask=match & (iota == k))
        return carry

    lax.fori_loop(0, _V // _LANES, zero_body, 0)

    @pl.when(m > 0)
    def _():
        lax.fori_loop(0, _CHUNKS, scat_body, 0)

    pltpu.sync_copy(cnt_v, cnt_hbm.at[b])


def _tc_norm_body(alpha_ref, m_ref, cnt_ref, out_ref):
    alpha = alpha_ref[0]
    denom = alpha * jnp.float32(_V) + m_ref[...]
    out_ref[...] = jnp.log(cnt_ref[...] + alpha) - jnp.log(denom)


_ROWS_PER_BLK = 8

_tc_norm = pl.pallas_call(
    _tc_norm_body,
    grid=(_B // _ROWS_PER_BLK,),
    in_specs=[
        pl.BlockSpec(memory_space=pltpu.SMEM),
        pl.BlockSpec((_ROWS_PER_BLK, 1), lambda i: (i, 0)),
        pl.BlockSpec((_ROWS_PER_BLK, _V), lambda i: (i, 0)),
    ],
    out_specs=pl.BlockSpec((_ROWS_PER_BLK, _V), lambda i: (i, 0)),
    out_shape=jax.ShapeDtypeStruct((_B, _V), jnp.float32),
)


def kernel(input_ids, alpha):
    ids = input_ids.astype(jnp.int32)
    t1b = jnp.broadcast_to(ids[:, _L - 2 : _L - 1], (_B, _LANES))
    t2b = jnp.broadcast_to(ids[:, _L - 1 : _L], (_B, _LANES))

    counts, m_splat = _sc_hist(ids, t1b, t2b)
    mcol = m_splat[:, :1]

    alpha1 = jnp.reshape(alpha, (1,)).astype(jnp.float32)
    logits = _tc_norm(alpha1, mcol, counts)
    return logits.reshape(_B, 1, _V)


# speedup vs baseline: 9.8067x; 1.8073x over previous
import functools

import jax
import jax.numpy as jnp
from jax import lax
from jax.experimental import pallas as pl
from jax.experimental.pallas import tpu as pltpu
from jax.experimental.pallas import tpu_sc as plsc

_N = 3
_V = 100000
_B = 32
_L = 4096
_W = _L - _N + 1
_LANES = 16
_NC, _NS = 2, 16
_CHUNKS = (_W + _LANES - 1) // _LANES
_ROWBUF = _CHUNKS * _LANES + _LANES

_mesh = plsc.VectorSubcoreMesh(
    core_axis_name="c", subcore_axis_name="s", num_cores=_NC, num_subcores=_NS
)


@functools.partial(
    pl.kernel,
    out_type=[
        jax.ShapeDtypeStruct((_B, _V), jnp.float32),
        jax.ShapeDtypeStruct((_B, _LANES), jnp.float32),
    ],
    mesh=_mesh,
    scratch_types=[
        pltpu.VMEM((_ROWBUF,), jnp.int32),
        pltpu.VMEM((_V,), jnp.float32),
        pltpu.VMEM((_LANES,), jnp.int32),
        pltpu.VMEM((_LANES,), jnp.int32),
        pltpu.VMEM((_LANES,), jnp.float32),
        pltpu.VMEM((_LANES,), jnp.int32),
    ],
    compiler_params=pltpu.CompilerParams(needs_layout_passes=False),
)
def _sc_hist(ids_hbm, t1_hbm, t2_hbm, cnt_hbm, m_hbm, row_v, cnt_v, t1_v, t2_v, m_v, macc_v):
    b = lax.axis_index("s") * _NC + lax.axis_index("c")

    pltpu.sync_copy(ids_hbm.at[b], row_v.at[pl.ds(0, _L)])
    pltpu.sync_copy(t1_hbm.at[b], t1_v)
    pltpu.sync_copy(t2_hbm.at[b], t2_v)

    t1 = t1_v[...]
    t2 = t2_v[...]
    iota = lax.broadcasted_iota(jnp.int32, (_LANES,), 0)

    def _match(base):
        a = row_v[pl.ds(base, _LANES)]
        bb = row_v[pl.ds(base + 1, _LANES)]
        return (a == t1) & (bb == t2) & ((base + iota) < _W)

    def count_body(i, acc):
        return acc + plsc.all_reduce_population_count(_match(i * _LANES))

    acc = lax.fori_loop(0, _CHUNKS, count_body, jnp.zeros((_LANES,), jnp.int32))

    macc_v[...] = acc
    m = acc[0]
    m_v[...] = acc.astype(jnp.float32)
    pltpu.sync_copy(m_v, m_hbm.at[b])

    zeros16 = jnp.zeros((_LANES,), jnp.float32)
    ones16 = jnp.ones((_LANES,), jnp.float32)

    def zero_body(j, carry):
        cnt_v[pl.ds(j * _LANES, _LANES)] = zeros16
        return carry

    def scat_body(i, carry):
        base = i * _LANES
        match = _match(base)
        nx = jnp.where(match, row_v[pl.ds(base + 2, _LANES)], 0)
        for k in range(_LANES):
            plsc.addupdate_scatter(cnt_v, [nx], ones16, mask=match & (iota == k))
        return carry

    lax.fori_loop(0, _V // _LANES, zero_body, 0)

    @pl.when(m > 0)
    def _():
        lax.fori_loop(0, _CHUNKS, scat_body, 0)

    pltpu.sync_copy(cnt_v, cnt_hbm.at[b])


@functools.partial(
    pl.kernel,
    out_type=jax.ShapeDtypeStruct((_B, _LANES), jnp.float32),
    mesh=_mesh,
    scratch_types=[
        pltpu.VMEM((_ROWBUF,), jnp.int32),
        pltpu.VMEM((_LANES,), jnp.float32),
    ],
    compiler_params=pltpu.CompilerParams(needs_layout_passes=False),
)
def _sc_scan(ids_hbm, m_hbm, row_v, m_v):
    b = lax.axis_index("s") * _NC + lax.axis_index("c")
    pltpu.sync_copy(ids_hbm.at[b], row_v.at[pl.ds(0, _L)])

    tail = row_v[pl.ds(_L - _LANES, _LANES)]
    t1 = jnp.full((_LANES,), tail[_LANES - 2])
    t2 = jnp.full((_LANES,), tail[_LANES - 1])
    iota = lax.broadcasted_iota(jnp.int32, (_LANES,), 0)

    def count_body(i, acc):
        base = i * _LANES
        a = row_v[pl.ds(base, _LANES)]
        bb = row_v[pl.ds(base + 1, _LANES)]
        return acc + plsc.all_reduce_population_count((a == t1) & (bb == t2))

    acc = lax.fori_loop(0, _CHUNKS - 1, count_body, jnp.zeros((_LANES,), jnp.int32))
    base = (_CHUNKS - 1) * _LANES
    a = row_v[pl.ds(base, _LANES)]
    bb = row_v[pl.ds(base + 1, _LANES)]
    tail_match = (a == t1) & (bb == t2) & ((base + iota) < _W)
    acc = acc + plsc.all_reduce_population_count(tail_match)

    m_v[...] = acc.astype(jnp.float32)
    pltpu.sync_copy(m_v, m_hbm.at[b])


def _tc_fill_body(alpha_ref, m_ref, out_ref):
    alpha = alpha_ref[0]
    denom = alpha * jnp.float32(_V) + m_ref[...]
    la = jnp.log(jnp.full((_ROWS_PER_BLK, 1), alpha, jnp.float32))
    out_ref[...] = lax.broadcast_in_dim(
        la - jnp.log(denom), (_ROWS_PER_BLK, _V), (0, 1)
    )


def _tc_norm_body(alpha_ref, m_ref, cnt_ref, out_ref):
    alpha = alpha_ref[0]
    denom = alpha * jnp.float32(_V) + m_ref[...]
    out_ref[...] = jnp.log(cnt_ref[...] + alpha) - jnp.log(denom)


_ROWS_PER_BLK = 8

_tc_fill = pl.pallas_call(
    _tc_fill_body,
    grid=(_B // _ROWS_PER_BLK,),
    in_specs=[
        pl.BlockSpec(memory_space=pltpu.SMEM),
        pl.BlockSpec((_ROWS_PER_BLK, 1), lambda i: (i, 0)),
    ],
    out_specs=pl.BlockSpec((_ROWS_PER_BLK, _V), lambda i: (i, 0)),
    out_shape=jax.ShapeDtypeStruct((_B, _V), jnp.float32),
)

_tc_norm = pl.pallas_call(
    _tc_norm_body,
    grid=(_B // _ROWS_PER_BLK,),
    in_specs=[
        pl.BlockSpec(memory_space=pltpu.SMEM),
        pl.BlockSpec((_ROWS_PER_BLK, 1), lambda i: (i, 0)),
        pl.BlockSpec((_ROWS_PER_BLK, _V), lambda i: (i, 0)),
    ],
    out_specs=pl.BlockSpec((_ROWS_PER_BLK, _V), lambda i: (i, 0)),
    out_shape=jax.ShapeDtypeStruct((_B, _V), jnp.float32),
)


def _full_path(ids, t1b, t2b, mcol, alpha1):
    counts, _ = _sc_hist(ids, t1b, t2b)
    return _tc_norm(alpha1, mcol, counts)


def _fill_path(ids, t1b, t2b, mcol, alpha1):
    del ids, t1b, t2b
    return _tc_fill(alpha1, mcol)


def kernel(input_ids, alpha):
    ids = input_ids.astype(jnp.int32)

    m_splat = _sc_scan(ids)
    mcol = m_splat[:, :1]
    alpha1 = jnp.reshape(alpha, (1,)).astype(jnp.float32)

    t1b = jnp.broadcast_to(ids[:, _L - 2 : _L - 1], (_B, _LANES))
    t2b = jnp.broadcast_to(ids[:, _L - 1 : _L], (_B, _LANES))

    any_match = jnp.max(m_splat) > 0.0
    logits = lax.cond(any_match, _full_path, _fill_path, ids, t1b, t2b, mcol, alpha1)
    return logits.reshape(_B, 1, _V)


# speedup vs baseline: 18.0471x; 1.8403x over previous
import functools

import jax
import jax.numpy as jnp
from jax import lax
from jax.experimental import pallas as pl
from jax.experimental.pallas import tpu as pltpu
from jax.experimental.pallas import tpu_sc as plsc

_N = 3
_V = 100000
_B = 32
_L = 4096
_W = _L - _N + 1
_LANES = 16
_NC, _NS = 2, 16
_CHUNKS = (_W + _LANES - 1) // _LANES
_ROWBUF = _CHUNKS * _LANES + _LANES

_mesh = plsc.VectorSubcoreMesh(
    core_axis_name="c", subcore_axis_name="s", num_cores=_NC, num_subcores=_NS
)


@functools.partial(
    pl.kernel,
    out_type=[
        jax.ShapeDtypeStruct((_B, 1, _V), jnp.float32),
        jax.ShapeDtypeStruct((_B, _LANES), jnp.float32),
    ],
    mesh=_mesh,
    scratch_types=[
        pltpu.VMEM((_ROWBUF,), jnp.int32),
        pltpu.VMEM((_V,), jnp.float32),
        pltpu.VMEM((_LANES,), jnp.float32),
    ],
    compiler_params=pltpu.CompilerParams(needs_layout_passes=False),
)
def _sc_ngram(ids_hbm, cnt_hbm, m_hbm, row_v, cnt_v, m_v):
    b = lax.axis_index("s") * _NC + lax.axis_index("c")

    pltpu.sync_copy(ids_hbm.at[b], row_v.at[pl.ds(0, _L)])

    tail = row_v[pl.ds(_L - _LANES, _LANES)]
    t1 = jnp.full((_LANES,), tail[_LANES - 2])
    t2 = jnp.full((_LANES,), tail[_LANES - 1])
    iota = lax.broadcasted_iota(jnp.int32, (_LANES,), 0)

    def _match(base):
        a = row_v[pl.ds(base, _LANES)]
        bb = row_v[pl.ds(base + 1, _LANES)]
        return (a == t1) & (bb == t2)

    def count_body(i, acc):
        base = i * (5 * _LANES)
        for u in range(5):
            acc = acc + plsc.all_reduce_population_count(_match(base + u * _LANES))
        return acc

    acc = lax.fori_loop(
        0, (_CHUNKS - 1) // 5, count_body, jnp.zeros((_LANES,), jnp.int32)
    )
    base0 = (_CHUNKS - 1) * _LANES
    tail_match = _match(base0) & ((base0 + iota) < _W)
    acc = acc + plsc.all_reduce_population_count(tail_match)
    m = acc[0]

    m_v[...] = acc.astype(jnp.float32)
    pltpu.sync_copy(m_v, m_hbm.at[b])

    zeros16 = jnp.zeros((_LANES,), jnp.float32)
    ones16 = jnp.ones((_LANES,), jnp.float32)

    def zero_body(j, carry):
        cnt_v[pl.ds(j * _LANES, _LANES)] = zeros16
        return carry

    def scat_body(i, carry):
        base = i * _LANES
        match = _match(base) & ((base + iota) < _W)
        nx = jnp.where(match, row_v[pl.ds(base + 2, _LANES)], 0)
        for k in range(_LANES):
            plsc.addupdate_scatter(cnt_v, [nx], ones16, mask=match & (iota == k))
        return carry

    @pl.when(m > 0)
    def _():
        lax.fori_loop(0, _V // _LANES, zero_body, 0)
        lax.fori_loop(0, _CHUNKS, scat_body, 0)
        pltpu.sync_copy(cnt_v, cnt_hbm.at[b, 0])


def _tc_fill_body(alpha_ref, ids_ref, out_ref):
    alpha = alpha_ref[0]
    ids = ids_ref[...]
    t1 = ids[:, _L - 2 : _L - 1]
    t2 = ids[:, _L - 1 : _L]
    match = (ids[:, 0:_W] == t1) & (ids[:, 1 : _W + 1] == t2)
    m32 = jnp.sum(match.astype(jnp.float32), axis=1, keepdims=True)
    denom = alpha * jnp.float32(_V) + m32
    ld3 = lax.broadcast_in_dim(jnp.log(denom), (_B, 1, _V), (0, 2))
    la = jnp.log(jnp.full((_B, 1), alpha, jnp.float32))
    out_ref[...] = lax.broadcast_in_dim(la, (_B, 1, _V), (0, 2)) - ld3


_tc_fill = pl.pallas_call(
    _tc_fill_body,
    in_specs=[
        pl.BlockSpec(memory_space=pltpu.SMEM),
        pl.BlockSpec((_B, _L), lambda: (0, 0)),
    ],
    out_specs=pl.BlockSpec((_B, 1, _V), lambda: (0, 0, 0)),
    out_shape=jax.ShapeDtypeStruct((_B, 1, _V), jnp.float32),
)


def _tc_patch_body(alpha_ref, m_ref, cnt_hbm, fill_hbm, out_hbm, buf_v, sem):
    del fill_hbm
    m32 = m_ref[:, 0:1]

    @pl.when(jnp.max(m32) > 0.0)
    def _():
        alpha = alpha_ref[0]
        cp = pltpu.make_async_copy(cnt_hbm, buf_v, sem)
        cp.start()
        cp.wait()
        denom = alpha * jnp.float32(_V) + m32
        ld3 = lax.broadcast_in_dim(jnp.log(denom), (_B, 1, _V), (0, 2))
        la = jnp.log(jnp.full((_B, 1), alpha, jnp.float32))
        fill = lax.broadcast_in_dim(la, (_B, 1, _V), (0, 2)) - ld3
        logs = jnp.log(buf_v[...] + alpha) - ld3
        no_match_row = lax.broadcast_in_dim(m32 == 0.0, (_B, 1, _V), (0, 1))
        buf_v[...] = jnp.where(no_match_row, fill, logs)
        cp2 = pltpu.make_async_copy(buf_v, out_hbm, sem)
        cp2.start()
        cp2.wait()


_tc_patch = pl.pallas_call(
    _tc_patch_body,
    in_specs=[
        pl.BlockSpec(memory_space=pltpu.SMEM),
        pl.BlockSpec((_B, _LANES), lambda: (0, 0)),
        pl.BlockSpec(memory_space=pl.ANY),
        pl.BlockSpec(memory_space=pl.ANY),
    ],
    out_specs=pl.BlockSpec(memory_space=pl.ANY),
    out_shape=jax.ShapeDtypeStruct((_B, 1, _V), jnp.float32),
    scratch_shapes=[
        pltpu.VMEM((_B, 1, _V), jnp.float32),
        pltpu.SemaphoreType.DMA,
    ],
    input_output_aliases={3: 0},
)


def kernel(input_ids, alpha):
    ids = input_ids.astype(jnp.int32)
    alpha1 = jnp.reshape(alpha, (1,)).astype(jnp.float32)

    counts, m_splat = _sc_ngram(ids)
    counts = pltpu.with_memory_space_constraint(counts, pltpu.MemorySpace.HBM)
    fill = _tc_fill(alpha1, ids)
    return _tc_patch(alpha1, m_splat, counts, fill)
